# Initial kernel scaffold; baseline (speedup 1.0000x reference)
import jax
import jax.numpy as jnp
from jax.experimental import pallas as pl


def kernel(x, edge_index, W1, b1, W2, b2, W3, b3, W4, b4):
    raise NotImplementedError("write your pallas kernel here")


# speedup vs baseline: 10.3291x; 10.3291x over previous
import functools

import jax
import jax.numpy as jnp
from jax import lax
from jax.experimental import pallas as pl
from jax.experimental.pallas import tpu as pltpu
from jax.experimental.pallas import tpu_sc as plsc

N = 10000
N_PAD = 10240
E_RAW = 320000
E = E_RAW + N
SC_CORES = 2
SC_SUBCORES = 16
LANES = 16
NW = SC_CORES * SC_SUBCORES
BLK = 128
BPW = 81
EPW = BPW * BLK
E_PAD = NW * EPW
RPT = N_PAD // SC_SUBCORES
ROW_BLK = 512
GRID = N_PAD // ROW_BLK

_MESH = plsc.VectorSubcoreMesh(core_axis_name="c", subcore_axis_name="s",
                               num_cores=SC_CORES, num_subcores=SC_SUBCORES)


def _deg_body(dst_hbm, out_hbm, dst_v, hist_v):
    c = lax.axis_index("c")
    s = lax.axis_index("s")
    wid = c * SC_SUBCORES + s
    pltpu.sync_copy(dst_hbm.at[wid], dst_v)
    zero16 = jnp.zeros((LANES,), jnp.float32)
    ones16 = jnp.ones((LANES,), jnp.float32)

    def zb(i, carry):
        hist_v[pl.ds(i * LANES, LANES)] = zero16
        return carry

    lax.fori_loop(0, N_PAD // LANES, zb, 0)

    def hb(i, carry):
        b = i // (BLK // LANES)
        k = i % (BLK // LANES)
        idx = dst_v[b, pl.ds(k * LANES, LANES)]
        plsc.addupdate_scatter(hist_v, [idx], ones16)
        return carry

    lax.fori_loop(0, BPW * (BLK // LANES), hb, 0)
    pltpu.sync_copy(hist_v, out_hbm.at[wid])


_SC_PARAMS = pltpu.CompilerParams(needs_layout_passes=False)

_sc_deg = pl.kernel(
    _deg_body,
    out_type=jax.ShapeDtypeStruct((NW, N_PAD), jnp.float32),
    mesh=_MESH,
    compiler_params=_SC_PARAMS,
    scratch_types=[
        pltpu.VMEM((BPW, BLK), jnp.int32),
        pltpu.VMEM((N_PAD,), jnp.float32),
    ],
)


def _make_spmm(nc):
    def body(src_hbm, dst_hbm, zeros_hbm, *rest):
        xs = rest[:nc]
        out_hbm = rest[nc]
        src_v, dst_v, gbuf, acc, sem = rest[nc + 1:]
        c = lax.axis_index("c")
        s = lax.axis_index("s")
        wid = c * SC_SUBCORES + s
        pltpu.sync_copy(src_hbm.at[wid], src_v)
        pltpu.sync_copy(dst_hbm.at[wid], dst_v)
        for j in range(nc):
            pltpu.sync_copy(zeros_hbm, acc.at[pl.ds(s * RPT, RPT), :])
            plsc.subcore_barrier()

            def blk(b, carry):
                pltpu.async_copy(xs[j].at[src_v.at[b]], gbuf, sem).wait()
                pltpu.sync_copy(gbuf, acc.at[dst_v.at[b]], add=True)
                return carry

            lax.fori_loop(0, BPW, blk, 0)
            plsc.subcore_barrier()
            pltpu.sync_copy(acc.at[pl.ds(s * RPT, RPT), :],
                            out_hbm.at[c, j, pl.ds(s * RPT, RPT), :])
            plsc.subcore_barrier()

    return pl.kernel(
        body,
        out_type=jax.ShapeDtypeStruct((SC_CORES, nc, N_PAD, 128), jnp.float32),
        mesh=_MESH,
        compiler_params=_SC_PARAMS,
        scratch_types=[
            pltpu.VMEM((BPW, BLK), jnp.int32),
            pltpu.VMEM((BPW, BLK), jnp.int32),
            pltpu.VMEM((BLK, 128), jnp.float32),
            pltpu.VMEM_SHARED((N_PAD, 128), jnp.float32),
            pltpu.SemaphoreType.DMA,
        ],
    )


_sc_spmm1 = _make_spmm(1)
_sc_spmm4 = _make_spmm(4)


def _gelu(x):
    return 0.5 * x * (1.0 + lax.erf(x * 0.7071067811865476))


def _tc_a_body(degp_ref, x_ref, dinv_ref, x1s_ref):
    deg = jnp.sum(degp_ref[...], axis=0)
    dinv = jnp.where(deg > 0, lax.rsqrt(jnp.maximum(deg, 1e-12)), 0.0)
    dinv_ref[...] = dinv
    x1s_ref[...] = x_ref[...] * dinv[:, None]


def _tc_a(degp, x_pad):
    return pl.pallas_call(
        _tc_a_body,
        grid=(GRID,),
        in_specs=[
            pl.BlockSpec((NW, ROW_BLK), lambda i: (0, i)),
            pl.BlockSpec((ROW_BLK, 128), lambda i: (i, 0)),
        ],
        out_specs=[
            pl.BlockSpec((ROW_BLK,), lambda i: (i,)),
            pl.BlockSpec((ROW_BLK, 128), lambda i: (i, 0)),
        ],
        out_shape=[
            jax.ShapeDtypeStruct((N_PAD,), jnp.float32),
            jax.ShapeDtypeStruct((N_PAD, 128), jnp.float32),
        ],
    )(degp, x_pad)


def _sum_partials(p_ref, nc):
    g = p_ref[0] + p_ref[1]
    return jnp.concatenate([g[j] for j in range(nc)], axis=1)


def _tc_b_body(p_ref, dinv_ref, w1_ref, b1_ref, *out_refs):
    g = _sum_partials(p_ref, 1)
    dinv = dinv_ref[...]
    h = jnp.dot(g * dinv[:, None], w1_ref[...],
                preferred_element_type=jnp.float32) + b1_ref[...]
    h = _gelu(h) * dinv[:, None]
    for j in range(4):
        out_refs[j][...] = h[:, j * 128:(j + 1) * 128]


def _tc_b(p1, dinv, W1, b1):
    return pl.pallas_call(
        _tc_b_body,
        grid=(GRID,),
        in_specs=[
            pl.BlockSpec((SC_CORES, 1, ROW_BLK, 128), lambda i: (0, 0, i, 0)),
            pl.BlockSpec((ROW_BLK,), lambda i: (i,)),
            pl.BlockSpec((128, 512), lambda i: (0, 0)),
            pl.BlockSpec((512,), lambda i: (0,)),
        ],
        out_specs=[pl.BlockSpec((ROW_BLK, 128), lambda i: (i, 0))] * 4,
        out_shape=[jax.ShapeDtypeStruct((N_PAD, 128), jnp.float32)] * 4,
    )(p1, dinv, W1, b1)


def _tc_c_body(p_ref, dinv_ref, w2_ref, b2_ref, w3_ref, *out_refs):
    g = _sum_partials(p_ref, 4)
    dinv = dinv_ref[...]
    h = jnp.dot(g * dinv[:, None], w2_ref[...],
                preferred_element_type=jnp.float32) + b2_ref[...]
    h = _gelu(h)
    t = jnp.dot(h, w3_ref[...], preferred_element_type=jnp.float32)
    t = t * dinv[:, None]
    for j in range(4):
        out_refs[j][...] = t[:, j * 128:(j + 1) * 128]


def _tc_c(p2, dinv, W2, b2, W3):
    return pl.pallas_call(
        _tc_c_body,
        grid=(GRID,),
        in_specs=[
            pl.BlockSpec((SC_CORES, 4, ROW_BLK, 128), lambda i: (0, 0, i, 0)),
            pl.BlockSpec((ROW_BLK,), lambda i: (i,)),
            pl.BlockSpec((512, 768), lambda i: (0, 0)),
            pl.BlockSpec((768,), lambda i: (0,)),
            pl.BlockSpec((768, 512), lambda i: (0, 0)),
        ],
        out_specs=[pl.BlockSpec((ROW_BLK, 128), lambda i: (i, 0))] * 4,
        out_shape=[jax.ShapeDtypeStruct((N_PAD, 128), jnp.float32)] * 4,
    )(p2, dinv, W2, b2, W3)


def _tc_d_body(p_ref, dinv_ref, b3_ref, w4_ref, out_ref):
    g = _sum_partials(p_ref, 4)
    dinv = dinv_ref[...]
    h = _gelu(g * dinv[:, None] + b3_ref[...])
    t = jnp.dot(h, w4_ref[...], preferred_element_type=jnp.float32)
    out_ref[...] = t * dinv[:, None]


def _tc_d(p3, dinv, b3, W4):
    return pl.pallas_call(
        _tc_d_body,
        grid=(GRID,),
        in_specs=[
            pl.BlockSpec((SC_CORES, 4, ROW_BLK, 128), lambda i: (0, 0, i, 0)),
            pl.BlockSpec((ROW_BLK,), lambda i: (i,)),
            pl.BlockSpec((512,), lambda i: (0,)),
            pl.BlockSpec((512, 128), lambda i: (0, 0)),
        ],
        out_specs=pl.BlockSpec((ROW_BLK, 128), lambda i: (i, 0)),
        out_shape=jax.ShapeDtypeStruct((N_PAD, 128), jnp.float32),
    )(p3, dinv, b3, W4)


def _tc_e_body(p_ref, dinv_ref, b4_ref, out_ref):
    g = _sum_partials(p_ref, 1)
    out_ref[...] = g * dinv_ref[...][:, None] + b4_ref[...]


def _tc_e(p4, dinv, b4):
    return pl.pallas_call(
        _tc_e_body,
        grid=(GRID,),
        in_specs=[
            pl.BlockSpec((SC_CORES, 1, ROW_BLK, 128), lambda i: (0, 0, i, 0)),
            pl.BlockSpec((ROW_BLK,), lambda i: (i,)),
            pl.BlockSpec((128,), lambda i: (0,)),
        ],
        out_specs=pl.BlockSpec((ROW_BLK, 128), lambda i: (i, 0)),
        out_shape=jax.ShapeDtypeStruct((N_PAD, 128), jnp.float32),
    )(p4, dinv, b4)


@jax.jit
def _run(x, edge_index, W1, b1, W2, b2, W3, b3, W4, b4):
    ei = edge_index.astype(jnp.int32)
    loop = jnp.arange(N, dtype=jnp.int32)
    pad = jnp.full((E_PAD - E,), N, jnp.int32)
    src = jnp.concatenate([ei[0], loop, pad]).reshape(NW, BPW, BLK)
    dst = jnp.concatenate([ei[1], loop, pad]).reshape(NW, BPW, BLK)
    x_pad = jnp.pad(x, ((0, N_PAD - N), (0, 0)))
    ztile = jnp.zeros((RPT, 128), jnp.float32)

    degp = _sc_deg(dst)
    dinv, x1s = _tc_a(degp, x_pad)
    p1 = _sc_spmm1(src, dst, ztile, x1s)
    h1s = _tc_b(p1, dinv, W1, b1)
    p2 = _sc_spmm4(src, dst, ztile, *h1s)
    t3 = _tc_c(p2, dinv, W2, b2, W3)
    p3 = _sc_spmm4(src, dst, ztile, *t3)
    t4 = _tc_d(p3, dinv, b3, W4)
    p4 = _sc_spmm1(src, dst, ztile, t4)
    out = _tc_e(p4, dinv, b4)
    return out[:N]


def kernel(x, edge_index, W1, b1, W2, b2, W3, b3, W4, b4):
    return _run(x, edge_index, W1, b1, W2, b2, W3, b3, W4, b4)
